# baseline (speedup 1.0000x reference)
import jax
import jax.numpy as jnp
from jax import lax
from jax.experimental import pallas as pl
from jax.experimental.pallas import tpu as pltpu
from jax.experimental.pallas import tpu_sc as plsc

SEQ = 2048
BATCH = 4
D = 768
ROWS = SEQ * BATCH
NC, NS = 2, 16
NW = NC * NS
RPW = ROWS // NW
CHUNK = 64
NCHUNK = RPW // CHUNK
PE_CHUNK = CHUNK // BATCH
NBUF = 2
LANES = 16
DV = D // LANES


def _body(x_hbm, pe_hbm, table_hbm, out_hbm, idx_v, pe_v, rows_v,
          gsem, psem, ssem):
    wid = lax.axis_index("s") * NC + lax.axis_index("c")
    base = wid * RPW
    pbase = wid * (RPW // BATCH)

    pltpu.sync_copy(x_hbm.at[pl.ds(base, RPW)], idx_v)

    gathers = [None] * NCHUNK
    pes = [None] * NCHUNK
    stores = [None] * NCHUNK

    def issue(c):
        b = c % NBUF
        gathers[c] = pltpu.async_copy(
            table_hbm.at[idx_v.at[pl.ds(c * CHUNK, CHUNK)]],
            rows_v.at[b], gsem.at[b])
        pes[c] = pltpu.async_copy(
            pe_hbm.at[pl.ds(pbase + c * PE_CHUNK, PE_CHUNK), 0],
            pe_v.at[b], psem.at[b])

    for c in range(min(NBUF, NCHUNK)):
        issue(c)

    for c in range(NCHUNK):
        b = c % NBUF
        gathers[c].wait()
        pes[c].wait()

        def add_one(g, _, b=b, c=c):
            for k in range(DV):
                pv = pe_v[b, g, pl.ds(k * LANES, LANES)]
                for r in range(BATCH):
                    plsc.addupdate(
                        rows_v.at[b, g * BATCH + r, pl.ds(k * LANES, LANES)],
                        pv)
            pltpu.async_copy(
                rows_v.at[b, pl.ds(g * BATCH, BATCH)],
                out_hbm.at[pbase + c * PE_CHUNK + g],
                ssem.at[b])
            return 0

        lax.fori_loop(0, PE_CHUNK, add_one, 0, unroll=4)
        stores[c] = [
            pltpu.make_async_copy(
                rows_v.at[b, pl.ds(0, BATCH)],
                out_hbm.at[pbase],
                ssem.at[b])
            for _ in range(PE_CHUNK)]
        if c + NBUF < NCHUNK:
            for st in stores[c]:
                st.wait()
            issue(c + NBUF)
    for c in range(max(0, NCHUNK - NBUF), NCHUNK):
        for st in stores[c]:
            st.wait()


def kernel(x, table, pe):
    xf = x.reshape(ROWS).astype(jnp.int32)
    mesh = plsc.VectorSubcoreMesh(core_axis_name="c", subcore_axis_name="s")
    run = pl.kernel(
        _body,
        out_type=jax.ShapeDtypeStruct((SEQ, BATCH, D), jnp.float32),
        mesh=mesh,
        scratch_types=[
            pltpu.VMEM((RPW,), jnp.int32),
            pltpu.VMEM((NBUF, PE_CHUNK, D), jnp.float32),
            pltpu.VMEM((NBUF, CHUNK, D), jnp.float32),
            pltpu.SemaphoreType.DMA((NBUF,)),
            pltpu.SemaphoreType.DMA((NBUF,)),
            pltpu.SemaphoreType.DMA((NBUF,)),
        ],
    )
    return run(xf, pe, table)


# speedup vs baseline: 1.0876x; 1.0876x over previous
import jax
import jax.numpy as jnp
from jax import lax
from jax.experimental import pallas as pl
from jax.experimental.pallas import tpu as pltpu
from jax.experimental.pallas import tpu_sc as plsc

SEQ = 2048
BATCH = 4
D = 768
ROWS = SEQ * BATCH
NC, NS = 2, 16
NW = NC * NS
RPW = ROWS // NW
CHUNK = 64
NCHUNK = RPW // CHUNK
PE_CHUNK = CHUNK // BATCH
NBUF = 2
LANES = 16
DV = D // LANES


def _body(x_hbm, pe_hbm, table_hbm, out_hbm, idx_v, pe_v, rows_v,
          gsem, psem, ssem):
    wid = lax.axis_index("s") * NC + lax.axis_index("c")
    base = wid * RPW
    pbase = wid * (RPW // BATCH)

    pltpu.sync_copy(x_hbm.at[pl.ds(base, RPW)], idx_v)

    gathers = [None] * NCHUNK
    pes = [None] * NCHUNK
    stores = [None] * NCHUNK

    def issue(c):
        b = c % NBUF
        gathers[c] = pltpu.async_copy(
            table_hbm.at[idx_v.at[pl.ds(c * CHUNK, CHUNK)]],
            rows_v.at[b], gsem.at[b])
        pes[c] = pltpu.async_copy(
            pe_hbm.at[pl.ds(pbase + c * PE_CHUNK, PE_CHUNK), 0],
            pe_v.at[b], psem.at[b])

    for c in range(min(NBUF, NCHUNK)):
        issue(c)

    for c in range(NCHUNK):
        b = c % NBUF
        gathers[c].wait()
        pes[c].wait()

        def add_one(g, _, b=b, c=c):
            for k in range(DV):
                pv = pe_v[b, g, pl.ds(k * LANES, LANES)]
                for r in range(BATCH):
                    plsc.addupdate(
                        rows_v.at[b, g * BATCH + r, pl.ds(k * LANES, LANES)],
                        pv)
            pltpu.async_copy(
                rows_v.at[b, pl.ds(g * BATCH, BATCH)],
                out_hbm.at[pbase + c * PE_CHUNK + g],
                ssem.at[b])
            return 0

        lax.fori_loop(0, PE_CHUNK, add_one, 0, unroll=2)
        stores[c] = [
            pltpu.make_async_copy(
                rows_v.at[b, pl.ds(0, BATCH)],
                out_hbm.at[pbase],
                ssem.at[b])
            for _ in range(PE_CHUNK)]
        if c + NBUF < NCHUNK:
            for st in stores[c]:
                st.wait()
            issue(c + NBUF)
    for c in range(max(0, NCHUNK - NBUF), NCHUNK):
        for st in stores[c]:
            st.wait()


def kernel(x, table, pe):
    xf = x.reshape(ROWS).astype(jnp.int32)
    mesh = plsc.VectorSubcoreMesh(core_axis_name="c", subcore_axis_name="s")
    run = pl.kernel(
        _body,
        out_type=jax.ShapeDtypeStruct((SEQ, BATCH, D), jnp.float32),
        mesh=mesh,
        scratch_types=[
            pltpu.VMEM((RPW,), jnp.int32),
            pltpu.VMEM((NBUF, PE_CHUNK, D), jnp.float32),
            pltpu.VMEM((NBUF, CHUNK, D), jnp.float32),
            pltpu.SemaphoreType.DMA((NBUF,)),
            pltpu.SemaphoreType.DMA((NBUF,)),
            pltpu.SemaphoreType.DMA((NBUF,)),
        ],
    )
    return run(xf, pe, table)
